# baseline (speedup 1.0000x reference)
import functools

import jax
import jax.numpy as jnp
from jax import lax
from jax.experimental import pallas as pl
from jax.experimental.pallas import tpu as pltpu
from jax.experimental.pallas import tpu_sc as plsc

H = 8
N = 10000
E1 = 330000
N_PAD = 10112
TRASH = N_PAD - N
SUB = 128
BLK = 1024
NSUB = BLK // SUB
NBLK = 11
EC = BLK * NBLK
E_PAD = 32 * EC
ACC = H * N_PAD
ACC_T = ACC // 16
PB = 128
ECB = E_PAD // 16
NPB = ECB // PB
ROWS_T = N_PAD // 16

_mesh = plsc.VectorSubcoreMesh(core_axis_name="c", subcore_axis_name="s")
_params = pltpu.CompilerParams(needs_layout_passes=False)


@functools.partial(
    pl.kernel,
    mesh=_mesh,
    compiler_params=_params,
    out_type=[
        jax.ShapeDtypeStruct((2 * ACC,), jnp.float32),
    ],
    scratch_types=[
        pltpu.VMEM((EC,), jnp.int32),
        pltpu.VMEM((EC,), jnp.int32),
        pltpu.VMEM((N_PAD,), jnp.float32),
        pltpu.VMEM((N_PAD,), jnp.float32),
        pltpu.VMEM((BLK,), jnp.float32),
        pltpu.VMEM((NSUB, SUB), jnp.int32),
        pltpu.VMEM((ACC_T,), jnp.float32),
        pltpu.VMEM_SHARED((ACC,), jnp.float32),
    ],
)
def _phase_a(as_t, ad_t, s_all_h, d_all_h, den_out,
             s_all, d_all, as_h, ad_h, wbuf, idxb, zbuf, acc):
    c = lax.axis_index("c")
    sid = lax.axis_index("s")
    wid = sid * 2 + c

    pltpu.sync_copy(s_all_h.at[pl.ds(wid * EC, EC)], s_all)
    pltpu.sync_copy(d_all_h.at[pl.ds(wid * EC, EC)], d_all)

    def zloop(i, carry):
        zbuf[pl.ds(i * 16, 16)] = jnp.zeros((16,), jnp.float32)
        return carry
    lax.fori_loop(0, ACC_T // 16, zloop, 0)
    pltpu.sync_copy(zbuf, acc.at[pl.ds(sid * ACC_T, ACC_T)])
    plsc.subcore_barrier()

    for h in range(H):
        pltpu.sync_copy(as_t.at[pl.ds(h * N_PAD, N_PAD)], as_h)
        pltpu.sync_copy(ad_t.at[pl.ds(h * N_PAD, N_PAD)], ad_h)

        def blk(b, carry):
            base = b * BLK
            for j in range(NSUB):
                def chunk(k, cc):
                    off = base + j * SUB + k * 16
                    s_vec = s_all[pl.ds(off, 16)]
                    d_vec = d_all[pl.ds(off, 16)]
                    sv = plsc.load_gather(as_h, [s_vec])
                    dv = plsc.load_gather(ad_h, [d_vec])
                    v = sv + dv
                    v = jnp.maximum(v, v * jnp.float32(0.2))
                    wbuf[pl.ds(j * SUB + k * 16, 16)] = jnp.exp(v)
                    idxb[j, pl.ds(k * 16, 16)] = d_vec + h * N_PAD
                    return cc
                lax.fori_loop(0, SUB // 16, chunk, 0)
            for j in range(NSUB):
                pltpu.sync_copy(wbuf.at[pl.ds(j * SUB, SUB)],
                                acc.at[idxb.at[j]], add=True)
            return carry
        lax.fori_loop(0, NBLK, blk, 0)

    plsc.subcore_barrier()
    pltpu.sync_copy(acc.at[pl.ds(sid * ACC_T, ACC_T)], zbuf)
    pltpu.sync_copy(zbuf, den_out.at[pl.ds(c * ACC + sid * ACC_T, ACC_T)])


TBL = 3 * ACC
TBL_T = TBL // 16
STG = 1024


@functools.partial(
    pl.kernel,
    mesh=_mesh,
    compiler_params=_params,
    out_type=[
        jax.ShapeDtypeStruct((H * E_PAD,), jnp.float32),
        jax.ShapeDtypeStruct((4 * N_PAD, 128), jnp.float32),
    ],
    scratch_types=[
        pltpu.VMEM((STG,), jnp.float32),
        pltpu.VMEM((PB, 128), jnp.float32),
        pltpu.VMEM((PB,), jnp.int32),
        pltpu.VMEM((PB,), jnp.int32),
        pltpu.VMEM((6 * PB,), jnp.int32),
        pltpu.VMEM((6 * PB,), jnp.float32),
        pltpu.VMEM((1, PB), jnp.int32),
        pltpu.VMEM((1, PB), jnp.int32),
        pltpu.VMEM((PB + 16,), jnp.float32),
        pltpu.VMEM((PB + 16,), jnp.float32),
        pltpu.VMEM_SHARED((TBL,), jnp.float32),
        pltpu.VMEM_SHARED((N_PAD, 128), jnp.float32),
    ],
)
def _phase_b(xp_all, tbl_h, s_all_h, d_all_h,
             alpha_out, u_out,
             stage, xpb, s_b, d_b, idx6, val6, sidx, didx, al0, al1,
             tbl, acc):
    c = lax.axis_index("c")
    sid = lax.axis_index("s")
    wid = sid * 2 + c

    for k in range(TBL_T // STG + 1):
        ln = STG if (k + 1) * STG <= TBL_T else TBL_T - k * STG
        if ln <= 0:
            break
        off = sid * TBL_T + k * STG
        pltpu.sync_copy(tbl_h.at[pl.ds(off, ln)], stage.at[pl.ds(0, ln)])
        pltpu.sync_copy(stage.at[pl.ds(0, ln)], tbl.at[pl.ds(off, ln)])

    for pp in range(2):
        p = 2 * c + pp
        h0 = 2 * p

        def z16(i, carry):
            xpb[i // 8, pl.ds((i % 8) * 16, 16)] = jnp.zeros((16,),
                                                             jnp.float32)
            return carry
        lax.fori_loop(0, 64 * 8, z16, 0)
        for k in range(9):
            pltpu.sync_copy(xpb.at[pl.ds(0, 64)],
                            acc.at[pl.ds(sid * ROWS_T + k * 64, 64)])
        pltpu.sync_copy(xpb.at[pl.ds(0, 56)],
                        acc.at[pl.ds(sid * ROWS_T + 576, 56)])
        plsc.subcore_barrier()

        def batch(bt, carry):
            goff = sid * ECB + bt * PB
            pltpu.sync_copy(s_all_h.at[pl.ds(goff, PB)], s_b)
            pltpu.sync_copy(d_all_h.at[pl.ds(goff, PB)], d_b)

            def bld(k, cc):
                sl = pl.ds(k * 16, 16)
                s_vec = s_b[sl]
                d_vec = d_b[sl]
                o = k * 16
                idx6[pl.ds(o, 16)] = s_vec + h0 * N_PAD
                idx6[pl.ds(PB + o, 16)] = s_vec + (h0 + 1) * N_PAD
                idx6[pl.ds(2 * PB + o, 16)] = d_vec + (ACC + h0 * N_PAD)
                idx6[pl.ds(3 * PB + o, 16)] = d_vec + (ACC + (h0 + 1) * N_PAD)
                idx6[pl.ds(4 * PB + o, 16)] = d_vec + (2 * ACC + h0 * N_PAD)
                idx6[pl.ds(5 * PB + o, 16)] = (d_vec
                                               + (2 * ACC + (h0 + 1) * N_PAD))
                sidx[0, sl] = s_vec + p * N_PAD
                didx[0, sl] = d_vec
                return cc
            lax.fori_loop(0, PB // 16, bld, 0)

            pltpu.sync_copy(tbl.at[idx6], val6)
            pltpu.sync_copy(xp_all.at[sidx.at[0]], xpb)

            def chunk(k, cc):
                sl = pl.ds(k * 16, 16)
                o = k * 16
                v0 = val6[pl.ds(o, 16)] + val6[pl.ds(2 * PB + o, 16)]
                v0 = jnp.maximum(v0, v0 * jnp.float32(0.2))
                v1 = val6[pl.ds(PB + o, 16)] + val6[pl.ds(3 * PB + o, 16)]
                v1 = jnp.maximum(v1, v1 * jnp.float32(0.2))
                al0[sl] = jnp.exp(v0) / val6[pl.ds(4 * PB + o, 16)]
                al1[sl] = jnp.exp(v1) / val6[pl.ds(5 * PB + o, 16)]
                return cc
            lax.fori_loop(0, PB // 16, chunk, 0)

            pltpu.sync_copy(
                al0.at[pl.ds(0, PB)],
                alpha_out.at[pl.ds(h0 * E_PAD + goff, PB)])
            pltpu.sync_copy(
                al1.at[pl.ds(0, PB)],
                alpha_out.at[pl.ds((h0 + 1) * E_PAD + goff, PB)])

            def edge(e2, cc):
                va = jnp.full((16,), al0[pl.ds(e2, 16)][0], jnp.float32)
                vb = jnp.full((16,), al1[pl.ds(e2, 16)][0], jnp.float32)
                for cc4 in range(4):
                    sl = pl.ds(cc4 * 16, 16)
                    xpb[e2, sl] = xpb[e2, sl] * va
                for cc4 in range(4, 8):
                    sl = pl.ds(cc4 * 16, 16)
                    xpb[e2, sl] = xpb[e2, sl] * vb
                return cc
            lax.fori_loop(0, PB, edge, 0)

            pltpu.sync_copy(xpb, acc.at[didx.at[0]], add=True)
            return carry
        lax.fori_loop(0, NPB, batch, 0)
        plsc.subcore_barrier()

        for k in range(9):
            pltpu.sync_copy(acc.at[pl.ds(sid * ROWS_T + k * 64, 64)],
                            xpb.at[pl.ds(0, 64)])
            pltpu.sync_copy(
                xpb.at[pl.ds(0, 64)],
                u_out.at[pl.ds(p * N_PAD + sid * ROWS_T + k * 64, 64)])
        pltpu.sync_copy(acc.at[pl.ds(sid * ROWS_T + 576, 56)],
                        xpb.at[pl.ds(0, 56)])
        pltpu.sync_copy(
            xpb.at[pl.ds(0, 56)],
            u_out.at[pl.ds(p * N_PAD + sid * ROWS_T + 576, 56)])
        plsc.subcore_barrier()


NB = 128
NG = N_PAD // NB


def _pre_kernel(x_ref, w_ref, am_ref, dm_ref, xp4_ref, as8_ref, ad8_ref):
    xp_b = jnp.dot(x_ref[...], w_ref[...],
                   preferred_element_type=jnp.float32)
    for p in range(4):
        xp4_ref[p] = xp_b[:, p * 128:(p + 1) * 128]
    as8_ref[...] = lax.dot_general(
        am_ref[...], xp_b, (((0,), (1,)), ((), ())),
        preferred_element_type=jnp.float32)
    ad8_ref[...] = lax.dot_general(
        dm_ref[...], xp_b, (((0,), (1,)), ((), ())),
        preferred_element_type=jnp.float32)


def _dense_pre(x_pad, W, a_src, a_dst):
    in_dim = x_pad.shape[1]
    amat = jnp.zeros((512, H), jnp.float32)
    dmat = jnp.zeros((512, H), jnp.float32)
    av = a_src.reshape(H, 64)
    dv = a_dst.reshape(H, 64)
    for h in range(H):
        amat = amat.at[h * 64:(h + 1) * 64, h].set(av[h])
        dmat = dmat.at[h * 64:(h + 1) * 64, h].set(dv[h])
    return pl.pallas_call(
        _pre_kernel,
        grid=(NG,),
        in_specs=[
            pl.BlockSpec((NB, in_dim), lambda i: (i, 0)),
            pl.BlockSpec((in_dim, 512), lambda i: (0, 0)),
            pl.BlockSpec((512, H), lambda i: (0, 0)),
            pl.BlockSpec((512, H), lambda i: (0, 0)),
        ],
        out_specs=[
            pl.BlockSpec((4, NB, 128), lambda i: (0, i, 0)),
            pl.BlockSpec((H, NB), lambda i: (0, i)),
            pl.BlockSpec((H, NB), lambda i: (0, i)),
        ],
        out_shape=[
            jax.ShapeDtypeStruct((4, N_PAD, 128), jnp.float32),
            jax.ShapeDtypeStruct((H, N_PAD), jnp.float32),
            jax.ShapeDtypeStruct((H, N_PAD), jnp.float32),
        ],
    )(x_pad, W, amat, dmat)


def _den_kernel(d_ref, o_ref):
    o_ref[...] = d_ref[0] + d_ref[1] + jnp.float32(1e-16)


def _den_sum(den2):
    return pl.pallas_call(
        _den_kernel,
        out_shape=jax.ShapeDtypeStruct((ACC // 128, 128), jnp.float32),
    )(den2.reshape(2, ACC // 128, 128)).reshape(-1)


def _post_kernel(u_ref, b_ref, o_ref, *, act, i0):
    i = pl.program_id(0)
    acc = jnp.zeros((NB, 64), jnp.float32)
    for p in range(4):
        acc = acc + u_ref[p, :, :64] + u_ref[p, :, 64:]
    out = acc * jnp.float32(0.125) + b_ref[...]
    if act == "relu":
        rows = i * NB + lax.broadcasted_iota(jnp.int32, (NB, 64), 0)
        out = jnp.where(rows < i0, jnp.maximum(out, 0.0), 0.0)
    else:
        m = jnp.max(out, axis=-1, keepdims=True)
        e = jnp.exp(out - m)
        ssum = jnp.sum(e, axis=-1, keepdims=True)
        out = (out - m) - jnp.log(ssum)
    o_ref[...] = out


def _dense_post(u, b, act):
    import functools as _ft
    return pl.pallas_call(
        _ft.partial(_post_kernel, act=act, i0=N),
        grid=(NG,),
        in_specs=[
            pl.BlockSpec((4, NB, 128), lambda i: (0, i, 0)),
            pl.BlockSpec((1, 64), lambda i: (0, 0)),
        ],
        out_specs=pl.BlockSpec((NB, 64), lambda i: (i, 0)),
        out_shape=jax.ShapeDtypeStruct((N_PAD, 64), jnp.float32),
    )(u, b.reshape(1, 64))


def _gat_layer(x_pad, s_pad, d_pad, W, a_src, a_dst, b, act):
    xp4, as8, ad8 = _dense_pre(x_pad, W, a_src, a_dst)
    as_t = as8.reshape(-1)
    ad_t = ad8.reshape(-1)
    [den2] = _phase_a(as_t, ad_t, s_pad, d_pad)
    den_sum = _den_sum(den2)
    tbl_h = jnp.concatenate([as_t, ad_t, den_sum])
    xp_all = xp4.reshape(4 * N_PAD, 128)
    alpha_hm, u = _phase_b(xp_all, tbl_h, s_pad, d_pad)
    alpha = alpha_hm.reshape(H, E_PAD)[:, :E1].T
    out = _dense_post(u.reshape(4, N_PAD, 128), b, act)
    return out, alpha


def kernel(x, edge_index, W1, att_src1, att_dst1, b1,
           W2, att_src2, att_dst2, b2):
    n = x.shape[0]
    loops = jnp.arange(n, dtype=edge_index.dtype)
    ei = jnp.concatenate([edge_index, jnp.stack([loops, loops])], axis=1)
    s, d = ei[0], ei[1]
    padn = E_PAD - E1
    spread = (jnp.arange(padn, dtype=jnp.int32) % TRASH)
    s_pad = jnp.concatenate([s, spread])
    d_pad = jnp.concatenate([d, N + spread])
    x_pad = jnp.pad(x, ((0, TRASH), (0, 0)))
    h, alpha1 = _gat_layer(x_pad, s_pad, d_pad, W1, att_src1, att_dst1,
                           b1, "relu")
    out, alpha2 = _gat_layer(h, s_pad, d_pad, W2, att_src2, att_dst2,
                             b2, "lsm")
    return out[:n], ei, alpha1, alpha2
